# Initial kernel scaffold; baseline (speedup 1.0000x reference)
import jax
import jax.numpy as jnp
from jax.experimental import pallas as pl


def kernel(x, edge_index, W1, b1, W2, b2, W3, b3, gamma, beta, Wc, bc):
    raise NotImplementedError("write your pallas kernel here")


# speedup vs baseline: 13.6301x; 13.6301x over previous
import functools

import jax
import jax.numpy as jnp
from jax import lax
from jax.experimental import pallas as pl
from jax.experimental.pallas import tpu as pltpu
from jax.experimental.pallas import tpu_sc as plsc

N = 10000
NP = 10240
E = 320000
D = 128
DOUT = 16
EPS = 1e-5

NC = 2
NS = 16
NW = NC * NS
EPT = E // NW
K = 128
NFULL = EPT // K
KT = EPT - NFULL * K
RPT = NP // NS

_mesh = plsc.VectorSubcoreMesh(
    core_axis_name="c", subcore_axis_name="s", num_cores=NC, num_subcores=NS
)


def _zero_rows(buf, nrows, ncols):
    def body(i, _):
        for k in range(ncols // 16):
            buf[i, pl.ds(k * 16, 16)] = jnp.zeros((16,), jnp.float32)
        return 0

    lax.fori_loop(0, nrows, body, 0)


def _slab_writeout(acc_sh, out_hbm, c, base0, rows_v):
    for j in range(5):
        pltpu.sync_copy(acc_sh.at[pl.ds(base0 + j * K, K)], rows_v)
        pltpu.sync_copy(rows_v, out_hbm.at[c, pl.ds(base0 + j * K, K)])


@functools.partial(
    pl.kernel,
    out_type=jax.ShapeDtypeStruct((NC, NP, D), jnp.float32),
    mesh=_mesh,
    scratch_types=[
        pltpu.VMEM((K, D), jnp.float32),
        pltpu.VMEM((1, K), jnp.int32),
        pltpu.VMEM((1, 16), jnp.int32),
        pltpu.VMEM_SHARED((NP, D), jnp.float32),
    ],
)
def _deg(dst_hbm, out_hbm, ones_v, idx_v, idxt_v, acc_sh):
    c = lax.axis_index("c")
    s = lax.axis_index("s")
    wid = s * NC + c
    base0 = s * RPT

    _zero_rows(ones_v, K, D)
    for j in range(5):
        pltpu.sync_copy(ones_v, acc_sh.at[pl.ds(base0 + j * K, K)])
    plsc.subcore_barrier()

    def fill(i, _):
        for k in range(D // 16):
            ones_v[i, pl.ds(k * 16, 16)] = jnp.ones((16,), jnp.float32)
        return 0

    lax.fori_loop(0, K, fill, 0)

    def body(j, _):
        base = wid * EPT + j * K
        pltpu.sync_copy(dst_hbm.at[pl.ds(base, K)], idx_v.at[0])
        pltpu.sync_copy(ones_v, acc_sh.at[idx_v.at[0]], add=True)
        return 0

    lax.fori_loop(0, NFULL, body, 0)
    baset = wid * EPT + NFULL * K
    pltpu.sync_copy(dst_hbm.at[pl.ds(baset, KT)], idxt_v.at[0])
    pltpu.sync_copy(ones_v.at[pl.ds(0, KT), :], acc_sh.at[idxt_v.at[0]], add=True)

    plsc.subcore_barrier()
    _slab_writeout(acc_sh, out_hbm, c, base0, ones_v)


@functools.partial(
    pl.kernel,
    out_type=jax.ShapeDtypeStruct((NC, NP, D), jnp.float32),
    mesh=_mesh,
    scratch_types=[
        pltpu.VMEM((K, D), jnp.float32),
        pltpu.VMEM((1, K), jnp.int32),
        pltpu.VMEM((1, K), jnp.int32),
        pltpu.VMEM((1, 16), jnp.int32),
        pltpu.VMEM((1, 16), jnp.int32),
        pltpu.VMEM_SHARED((NP, D), jnp.float32),
        pltpu.SemaphoreType.DMA,
    ],
)
def _agg(g_hbm, src_hbm, dst_hbm, out_hbm, rows_v, sidx, didx, sidxt, didxt, acc_sh, sem):
    c = lax.axis_index("c")
    s = lax.axis_index("s")
    wid = s * NC + c
    base0 = s * RPT

    _zero_rows(rows_v, K, D)
    for j in range(5):
        pltpu.sync_copy(rows_v, acc_sh.at[pl.ds(base0 + j * K, K)])
    plsc.subcore_barrier()

    def body(j, _):
        base = wid * EPT + j * K
        pltpu.sync_copy(src_hbm.at[pl.ds(base, K)], sidx.at[0])
        pltpu.sync_copy(dst_hbm.at[pl.ds(base, K)], didx.at[0])
        pltpu.async_copy(g_hbm.at[sidx.at[0]], rows_v, sem).wait()
        pltpu.sync_copy(rows_v, acc_sh.at[didx.at[0]], add=True)
        return 0

    lax.fori_loop(0, NFULL, body, 0)
    baset = wid * EPT + NFULL * K
    pltpu.sync_copy(src_hbm.at[pl.ds(baset, KT)], sidxt.at[0])
    pltpu.sync_copy(dst_hbm.at[pl.ds(baset, KT)], didxt.at[0])
    pltpu.async_copy(g_hbm.at[sidxt.at[0]], rows_v.at[pl.ds(0, KT), :], sem).wait()
    pltpu.sync_copy(rows_v.at[pl.ds(0, KT), :], acc_sh.at[didxt.at[0]], add=True)

    plsc.subcore_barrier()
    _slab_writeout(acc_sh, out_hbm, c, base0, rows_v)


def _tc1_body(x_ref, w_ref, degs_ref, g_ref, dinv_ref):
    deg = degs_ref[0][0:N, 0:1] + degs_ref[1][0:N, 0:1] + 1.0
    dinv = lax.rsqrt(deg)
    dinv_ref[...] = dinv
    g_ref[...] = (
        jnp.dot(x_ref[...], w_ref[...], preferred_element_type=jnp.float32) * dinv
    )


def _tc_mid_body(acc_ref, g_ref, dinv_ref, b_ref, gamma_ref, beta_ref, w_ref, out_ref):
    dinv = dinv_ref[...]
    sacc = (acc_ref[0][0:N] + acc_ref[1][0:N] + g_ref[...]) * dinv + b_ref[...]
    h = jnp.maximum(sacc, 0.0)
    mean = jnp.mean(h, axis=0, keepdims=True)
    var = jnp.mean((h - mean) ** 2, axis=0, keepdims=True)
    hn = (h - mean) * lax.rsqrt(var + EPS) * gamma_ref[...] + beta_ref[...]
    out_ref[...] = (
        jnp.dot(hn, w_ref[...], preferred_element_type=jnp.float32) * dinv
    )


def _tc_fin_body(acc_ref, g_ref, dinv_ref, b_ref, wc_ref, bc_ref, out_ref):
    sacc = (acc_ref[0][0:N] + acc_ref[1][0:N] + g_ref[...]) * dinv_ref[...] + b_ref[...]
    h = jnp.maximum(sacc, 0.0)
    pooled = jnp.mean(h, axis=0, keepdims=True)
    out_ref[...] = (
        jnp.dot(pooled, wc_ref[...], preferred_element_type=jnp.float32) + bc_ref[...]
    )


_tc1 = pl.pallas_call(
    _tc1_body,
    out_shape=(
        jax.ShapeDtypeStruct((N, D), jnp.float32),
        jax.ShapeDtypeStruct((N, 1), jnp.float32),
    ),
)

_tc_mid = pl.pallas_call(
    _tc_mid_body,
    out_shape=jax.ShapeDtypeStruct((N, D), jnp.float32),
)

_tc_fin = pl.pallas_call(
    _tc_fin_body,
    out_shape=jax.ShapeDtypeStruct((1, DOUT), jnp.float32),
)


def kernel(x, edge_index, W1, b1, W2, b2, W3, b3, gamma, beta, Wc, bc):
    src = edge_index[0].astype(jnp.int32)
    dst = edge_index[1].astype(jnp.int32)
    b1r = b1.reshape(1, D)
    b2r = b2.reshape(1, D)
    b3r = b3.reshape(1, D)
    gr = gamma.reshape(1, D)
    br = beta.reshape(1, D)
    bcr = bc.reshape(1, DOUT)

    degs = _deg(dst)
    g1, dinv = _tc1(x, W1, degs)
    acc1 = _agg(g1, src, dst)
    g2 = _tc_mid(acc1, g1, dinv, b1r, gr, br, W2)
    acc2 = _agg(g2, src, dst)
    g3 = _tc_mid(acc2, g2, dinv, b2r, gr, br, W3)
    acc3 = _agg(g3, src, dst)
    return _tc_fin(acc3, g3, dinv, b3r, Wc, bcr)
